# baseline (speedup 1.0000x reference)
import functools
import math

import jax
import jax.numpy as jnp
from jax import lax
from jax.experimental import pallas as pl
from jax.experimental.pallas import tpu as pltpu
from jax.experimental.pallas import tpu_sc as plsc

_SMOOTHING = 0.1
_PAD = 1

_NC = 2
_NS = 16
_NW = _NC * _NS

_G = 25


def _sc_gather_body(lpf_hbm, tgt_hbm, vt_hbm, tgt_v, idx_v, val_v, sem, *, n):
    b = n // _NW
    wid = lax.axis_index("s") * _NC + lax.axis_index("c")
    base = wid * b
    pltpu.sync_copy(tgt_hbm.at[pl.ds(base, b)], tgt_v)
    for j in range(b // 16):
        t16 = tgt_v[pl.ds(j * 16, 16)]
        i16 = lax.iota(jnp.int32, 16) + (base + j * 16)
        idx_v[pl.ds(j * 16, 16)] = (
            (t16 >> 3) * (8 * n)
            + (i16 >> 7) * 1024
            + jnp.bitwise_and(t16, 7) * 128
            + jnp.bitwise_and(i16, 127)
        )
    pltpu.async_copy(lpf_hbm.at[idx_v], val_v, sem).wait()
    pltpu.sync_copy(val_v, vt_hbm.at[pl.ds(base, b)])


def _tc_a_body(tgt_ref, lpt_ref, out_ref, acc_s, vbrow, *, eps, conf, c, rv):
    s = pl.program_id(0)

    @pl.when(s == 0)
    def _():
        out_ref[0, 0] = 0.0
        acc_s[...] = jnp.zeros_like(acc_s)

    blk = lpt_ref[...]
    acc_s[...] += jnp.sum(blk, axis=0, keepdims=True)

    @pl.when(s == 0)
    def _():
        vbrow[...] = blk[_PAD:_PAD + 1, :]

    @pl.when(s == pl.num_programs(0) - 1)
    def _():
        tt = tgt_ref[...]
        m = tt != _PAD
        out_ref[0, 0] += jnp.sum(
            jnp.where(m, c - eps * acc_s[...] + eps * vbrow[...], 0.0)
        )


def _tc_c_body(p_ref, vt_ref, tgt_ref, out_ref, *, eps, conf):
    m = tgt_ref[...] != _PAD
    out_ref[0, 0] = p_ref[0, 0] + jnp.sum(
        jnp.where(m, (eps - conf) * vt_ref[...], 0.0)
    )


def kernel(log_probs, targets):
    lp = log_probs.reshape(-1, log_probs.shape[-1])
    n, v = lp.shape
    lpt = lp.T
    tgt = targets.reshape(-1).astype(jnp.int32)
    rv = v // _G
    eps = _SMOOTHING / (v - 2)
    conf = 1.0 - _SMOOTHING
    c = (v - 2) * eps * math.log(eps) + conf * math.log(conf)

    sc_gather = pl.kernel(
        functools.partial(_sc_gather_body, n=n),
        out_type=jax.ShapeDtypeStruct((n,), jnp.float32),
        mesh=plsc.VectorSubcoreMesh(core_axis_name="c", subcore_axis_name="s"),
        scratch_types=[
            pltpu.VMEM((n // _NW,), jnp.int32),
            pltpu.VMEM((n // _NW,), jnp.int32),
            pltpu.VMEM((n // _NW,), jnp.float32),
            pltpu.SemaphoreType.DMA,
        ],
    )
    p_a = pl.pallas_call(
        functools.partial(_tc_a_body, eps=eps, conf=conf, c=c, rv=rv),
        grid=(_G,),
        in_specs=[
            pl.BlockSpec((1, n), lambda i: (0, 0)),
            pl.BlockSpec((rv, n), lambda i: (i, 0)),
        ],
        out_specs=pl.BlockSpec(
            (1, 1), lambda i: (0, 0), memory_space=pltpu.SMEM
        ),
        out_shape=jax.ShapeDtypeStruct((1, 1), jnp.float32),
        scratch_shapes=[
            pltpu.VMEM((1, n), jnp.float32),
            pltpu.VMEM((1, n), jnp.float32),
        ],
    )(tgt.reshape(1, n), lpt)

    lpf = (
        lpt.reshape(v // 8, 8, n // 128, 128)
        .transpose(0, 2, 1, 3)
        .reshape(-1)
    )
    vt = sc_gather(lpf, tgt)

    out = pl.pallas_call(
        functools.partial(_tc_c_body, eps=eps, conf=conf),
        in_specs=[
            pl.BlockSpec(memory_space=pltpu.SMEM),
            pl.BlockSpec(memory_space=pltpu.VMEM),
            pl.BlockSpec(memory_space=pltpu.VMEM),
        ],
        out_specs=pl.BlockSpec(memory_space=pltpu.SMEM),
        out_shape=jax.ShapeDtypeStruct((1, 1), jnp.float32),
    )(p_a, vt.reshape(1, n), tgt.reshape(1, n))
    return out[0, 0]


# speedup vs baseline: 1.0402x; 1.0402x over previous
import functools
import math

import jax
import jax.numpy as jnp
from jax import lax
from jax.experimental import pallas as pl
from jax.experimental.pallas import tpu as pltpu
from jax.experimental.pallas import tpu_sc as plsc

_SMOOTHING = 0.1
_PAD = 1

_NC = 2
_NS = 16
_NW = _NC * _NS

_G = 25


def _sc_gather_body(lpf_hbm, tgt_hbm, vt_hbm, tgt_v, idx_v, val_v, sem, *, n):
    b = n // _NW
    wid = lax.axis_index("s") * _NC + lax.axis_index("c")
    base = wid * b
    pltpu.sync_copy(tgt_hbm.at[pl.ds(base, b)], tgt_v)
    for j in range(b // 16):
        t16 = tgt_v[pl.ds(j * 16, 16)]
        i16 = lax.iota(jnp.int32, 16) + (base + j * 16)
        idx_v[pl.ds(j * 16, 16)] = (
            (t16 >> 3) * (8 * n)
            + (i16 >> 7) * 1024
            + jnp.bitwise_and(t16, 7) * 128
            + jnp.bitwise_and(i16, 127)
        )
    pltpu.async_copy(lpf_hbm.at[idx_v], val_v, sem).wait()
    pltpu.sync_copy(val_v, vt_hbm.at[pl.ds(base, b)])


def _tc_a_body(tgt_ref, lpt_ref, out_ref, acc_s, vbrow, *, eps, conf, c, rv):
    s = pl.program_id(0)

    @pl.when(s == 0)
    def _():
        out_ref[0, 0] = 0.0
        acc_s[...] = jnp.zeros_like(acc_s)

    blk = lpt_ref[...]
    acc_s[...] += jnp.sum(blk, axis=0, keepdims=True)

    @pl.when(s == 0)
    def _():
        vbrow[...] = blk[_PAD:_PAD + 1, :]

    @pl.when(s == pl.num_programs(0) - 1)
    def _():
        tt = tgt_ref[...]
        m = tt != _PAD
        out_ref[0, 0] += jnp.sum(
            jnp.where(m, c - eps * acc_s[...] + eps * vbrow[...], 0.0)
        )


def _tc_c_body(p_ref, vt_ref, tgt_ref, out_ref, *, eps, conf):
    m = tgt_ref[...] != _PAD
    out_ref[0, 0] = p_ref[0, 0] + jnp.sum(
        jnp.where(m, (eps - conf) * vt_ref[...], 0.0)
    )


def kernel(log_probs, targets):
    lp = log_probs.reshape(-1, log_probs.shape[-1])
    n, v = lp.shape
    lpt = lp.T
    tgt = targets.reshape(-1).astype(jnp.int32)
    rv = v // _G
    eps = _SMOOTHING / (v - 2)
    conf = 1.0 - _SMOOTHING
    c = (v - 2) * eps * math.log(eps) + conf * math.log(conf)

    sc_gather = pl.kernel(
        functools.partial(_sc_gather_body, n=n),
        out_type=jax.ShapeDtypeStruct((n,), jnp.float32),
        mesh=plsc.VectorSubcoreMesh(core_axis_name="c", subcore_axis_name="s"),
        scratch_types=[
            pltpu.VMEM((n // _NW,), jnp.int32),
            pltpu.VMEM((n // _NW,), jnp.int32),
            pltpu.VMEM((n // _NW,), jnp.float32),
            pltpu.SemaphoreType.DMA,
        ],
    )
    lpf = (
        lpt.reshape(v // 8, 8, n // 128, 128)
        .transpose(0, 2, 1, 3)
        .reshape(-1)
    )
    vt = sc_gather(lpf, tgt)

    p_a = pl.pallas_call(
        functools.partial(_tc_a_body, eps=eps, conf=conf, c=c, rv=rv),
        grid=(_G,),
        in_specs=[
            pl.BlockSpec((1, n), lambda i: (0, 0)),
            pl.BlockSpec((rv, n), lambda i: (i, 0)),
        ],
        out_specs=pl.BlockSpec(
            (1, 1), lambda i: (0, 0), memory_space=pltpu.SMEM
        ),
        out_shape=jax.ShapeDtypeStruct((1, 1), jnp.float32),
        scratch_shapes=[
            pltpu.VMEM((1, n), jnp.float32),
            pltpu.VMEM((1, n), jnp.float32),
        ],
    )(tgt.reshape(1, n), lpt)

    out = pl.pallas_call(
        functools.partial(_tc_c_body, eps=eps, conf=conf),
        in_specs=[
            pl.BlockSpec(memory_space=pltpu.SMEM),
            pl.BlockSpec(memory_space=pltpu.VMEM),
            pl.BlockSpec(memory_space=pltpu.VMEM),
        ],
        out_specs=pl.BlockSpec(memory_space=pltpu.SMEM),
        out_shape=jax.ShapeDtypeStruct((1, 1), jnp.float32),
    )(p_a, vt.reshape(1, n), tgt.reshape(1, n))
    return out[0, 0]
